# baseline (speedup 1.0000x reference)
import functools

import jax
import jax.numpy as jnp
from jax import lax
from jax.experimental import pallas as pl
from jax.experimental.pallas import tpu as pltpu
from jax.experimental.pallas import tpu_sc as plsc

BOTTLENECK = 32
EMB = 128
NC, NS = 2, 16
NW = NC * NS
CH = 128
CPG = 5
GROUP = CH * CPG
SB = 32
BB = 4 * SB
LANES = 16


def _make_gather(B, n_batch, n_field):
    rows_per_w = B // NW
    n_chunks = rows_per_w // CH
    n_groups = n_chunks // CPG
    nat_rows = n_batch // NW
    chunks_per_j = BB * n_field // CH
    f_per_chunk = CH // (4 * SB)
    mesh = plsc.VectorSubcoreMesh(core_axis_name="c", subcore_axis_name="s")

    @functools.partial(
        pl.kernel,
        out_type=jax.ShapeDtypeStruct((B, BOTTLENECK), jnp.float32),
        mesh=mesh,
        scratch_types=[
            pltpu.VMEM((nat_rows, CH), jnp.int32),
            pltpu.VMEM((n_chunks, CH), jnp.int32),
            pltpu.VMEM((GROUP, BOTTLENECK), jnp.float32),
            pltpu.VMEM((GROUP, BOTTLENECK), jnp.float32),
            pltpu.SemaphoreType.DMA,
            pltpu.SemaphoreType.DMA,
        ],
        compiler_params=pltpu.CompilerParams(use_tc_tiling_on_sc=False,
                                             needs_layout_passes=False),
    )
    def gather_kernel(idx_hbm, tab_hbm, out_hbm, nat_v, perm_v, buf0, buf1,
                      sem0, sem1):
        wid = lax.axis_index("s") * NC + lax.axis_index("c")
        pltpu.sync_copy(idx_hbm.at[pl.ds(nat_rows * wid, nat_rows)], nat_v)
        row0 = rows_per_w * wid

        lanes = lax.iota(jnp.int32, LANES)
        bpat = []
        for v in range(4 * SB // LANES):
            r = LANES * v + lanes
            bpat.append(SB * (r % 4) + r // 4)

        def build(c, carry):
            j_loc = c // chunks_per_j
            fbase = (c % chunks_per_j) * f_per_chunk
            for i in range(CH // LANES):
                row = BB * j_loc + bpat[i % (4 * SB // LANES)]
                f = fbase + i // (4 * SB // LANES)
                col = jnp.full((LANES,), f, jnp.int32)
                perm_v[c, pl.ds(LANES * i, LANES)] = 4 * plsc.load_gather(
                    nat_v, [row, col])
            return carry

        lax.fori_loop(0, n_chunks, build, 0, unroll=False)

        def fire(g, buf, sem):
            for k in range(CPG):
                pltpu.async_copy(tab_hbm.at[perm_v.at[g * CPG + k]],
                                 buf.at[pl.ds(k * CH, CH)], sem)

        def drain_out(g, buf, sem):
            for k in range(CPG):
                pltpu.make_async_copy(tab_hbm.at[perm_v.at[g * CPG + k]],
                                      buf.at[pl.ds(k * CH, CH)], sem).wait()
            pltpu.sync_copy(buf, out_hbm.at[pl.ds(row0 + g * GROUP, GROUP)])

        fire(0, buf0, sem0)

        def body(i, carry):
            g = 2 * i
            fire(g + 1, buf1, sem1)
            drain_out(g, buf0, sem0)

            @pl.when(g + 2 < n_groups)
            def _():
                fire(g + 2, buf0, sem0)

            drain_out(g + 1, buf1, sem1)
            return carry

        lax.fori_loop(0, n_groups // 2, body, 0, unroll=False)

    return gather_kernel


def _pad_body(x_ref, o_ref):
    o_ref[:, 0:BOTTLENECK] = x_ref[...]


def _pad_table(table):
    rows8 = -(-table.shape[0] // 8) * 8
    blk = rows8 // 24
    return pl.pallas_call(
        _pad_body,
        grid=(rows8 // blk,),
        in_specs=[pl.BlockSpec((blk, BOTTLENECK), lambda i: (i, 0))],
        out_specs=pl.BlockSpec((blk, EMB), lambda i: (i, 0)),
        out_shape=jax.ShapeDtypeStruct((rows8, EMB), jnp.float32),
    )(table)


def _mm_body(x_ref, w_ref, o_ref):
    n_field = o_ref.shape[0]
    for k in range(4):
        x = x_ref[:, k * BOTTLENECK:(k + 1) * BOTTLENECK]
        y = jnp.dot(x, w_ref[...], preferred_element_type=jnp.float32)
        o_ref[:, pl.ds(k * SB, SB), :] = y.reshape(n_field, SB, EMB)


def _unpack_project(packed, W, n_batch, n_field):
    rows_per_block = SB * n_field
    m_grid = n_batch // BB
    return pl.pallas_call(
        _mm_body,
        grid=(m_grid,),
        in_specs=[
            pl.BlockSpec((rows_per_block, EMB), lambda m: (m, 0)),
            pl.BlockSpec((BOTTLENECK, EMB), lambda m: (0, 0)),
        ],
        out_specs=pl.BlockSpec((n_field, BB, EMB), lambda m: (0, m, 0)),
        out_shape=jax.ShapeDtypeStruct((n_field, n_batch, EMB), jnp.float32),
    )(packed, W)


def kernel(input, table, W):
    n_batch, n_field = input.shape
    B = n_batch * n_field
    idx = jnp.pad(input.astype(jnp.int32), ((0, 0), (0, CH - n_field)))
    table_pad = _pad_table(table)
    table_view = table_pad.reshape(4 * table_pad.shape[0], BOTTLENECK)
    flat_rows = _make_gather(B, n_batch, n_field)(idx, table_view)
    packed = flat_rows.reshape(B // 4, 4 * BOTTLENECK)
    out_fm = _unpack_project(packed, W, n_batch, n_field)
    return jnp.transpose(out_fm, (1, 0, 2))


# speedup vs baseline: 1.1085x; 1.1085x over previous
import functools

import jax
import jax.numpy as jnp
from jax import lax
from jax.experimental import pallas as pl
from jax.experimental.pallas import tpu as pltpu
from jax.experimental.pallas import tpu_sc as plsc

BOTTLENECK = 32
EMB = 128
NC, NS = 2, 16
NW = NC * NS
CH = 128
CPG = 5
GROUP = CH * CPG
SB = 32
BB = 4 * SB
LANES = 16


def _make_gather(B, n_field, b_off):
    rows_per_w = B // NW
    n_chunks = rows_per_w // CH
    n_groups = n_chunks // CPG
    chunks_per_j = BB * n_field // CH
    wpj = chunks_per_j // n_chunks
    f_per_chunk = CH // (4 * SB)
    mesh = plsc.VectorSubcoreMesh(core_axis_name="c", subcore_axis_name="s")

    @functools.partial(
        pl.kernel,
        out_type=jax.ShapeDtypeStruct((B, BOTTLENECK), jnp.float32),
        mesh=mesh,
        scratch_types=[
            pltpu.VMEM((BB, CH), jnp.int32),
            pltpu.VMEM((n_chunks, CH), jnp.int32),
            pltpu.VMEM((GROUP, BOTTLENECK), jnp.float32),
            pltpu.VMEM((GROUP, BOTTLENECK), jnp.float32),
            pltpu.SemaphoreType.DMA,
            pltpu.SemaphoreType.DMA,
        ],
        compiler_params=pltpu.CompilerParams(use_tc_tiling_on_sc=False,
                                             needs_layout_passes=False),
    )
    def gather_kernel(idx_hbm, tab_hbm, out_hbm, nat_v, perm_v, buf0, buf1,
                      sem0, sem1):
        wid = lax.axis_index("s") * NC + lax.axis_index("c")
        pltpu.sync_copy(idx_hbm.at[pl.ds(b_off + BB * (wid // wpj), BB)],
                        nat_v)
        row0 = rows_per_w * wid
        fchunk0 = (wid % wpj) * n_chunks

        lanes = lax.iota(jnp.int32, LANES)
        bpat = []
        for v in range(4 * SB // LANES):
            r = LANES * v + lanes
            bpat.append(SB * (r % 4) + r // 4)

        def build(c, carry):
            fbase = (fchunk0 + c) * f_per_chunk
            for i in range(CH // LANES):
                row = bpat[i % (4 * SB // LANES)]
                f = fbase + i // (4 * SB // LANES)
                col = jnp.full((LANES,), f, jnp.int32)
                perm_v[c, pl.ds(LANES * i, LANES)] = 4 * plsc.load_gather(
                    nat_v, [row, col])
            return carry

        lax.fori_loop(0, n_chunks, build, 0, unroll=False)

        def fire(g, buf, sem):
            for k in range(CPG):
                pltpu.async_copy(tab_hbm.at[perm_v.at[g * CPG + k]],
                                 buf.at[pl.ds(k * CH, CH)], sem)

        def drain_out(g, buf, sem):
            for k in range(CPG):
                pltpu.make_async_copy(tab_hbm.at[perm_v.at[g * CPG + k]],
                                      buf.at[pl.ds(k * CH, CH)], sem).wait()
            pltpu.sync_copy(buf, out_hbm.at[pl.ds(row0 + g * GROUP, GROUP)])

        fire(0, buf0, sem0)

        def body(i, carry):
            g = 2 * i
            fire(g + 1, buf1, sem1)
            drain_out(g, buf0, sem0)

            @pl.when(g + 2 < n_groups)
            def _():
                fire(g + 2, buf0, sem0)

            drain_out(g + 1, buf1, sem1)
            return carry

        lax.fori_loop(0, n_groups // 2, body, 0, unroll=False)

    return gather_kernel


def _mm_body(x_ref, w_ref, o_ref):
    n_field = o_ref.shape[0]
    for k in range(4):
        x = x_ref[:, k * BOTTLENECK:(k + 1) * BOTTLENECK]
        y = jnp.dot(x, w_ref[...], preferred_element_type=jnp.float32)
        o_ref[:, pl.ds(k * SB, SB), :] = y.reshape(n_field, SB, EMB)


def _mm_body_alias(x_ref, w_ref, a_ref, o_ref):
    _mm_body(x_ref, w_ref, o_ref)


def _unpack_project(packed, W, n_batch, n_field, half, prev=None):
    rows_per_block = SB * n_field
    m_grid = packed.shape[0] // rows_per_block
    base = half * m_grid
    out_shape = jax.ShapeDtypeStruct((n_field, n_batch, EMB), jnp.float32)
    in_specs = [
        pl.BlockSpec((rows_per_block, EMB), lambda m: (m, 0)),
        pl.BlockSpec((BOTTLENECK, EMB), lambda m: (0, 0)),
    ]
    out_spec = pl.BlockSpec((n_field, BB, EMB), lambda m: (0, m + base, 0))
    if prev is None:
        return pl.pallas_call(
            _mm_body, grid=(m_grid,), in_specs=in_specs,
            out_specs=out_spec, out_shape=out_shape,
        )(packed, W)
    return pl.pallas_call(
        _mm_body_alias, grid=(m_grid,),
        in_specs=in_specs + [pl.BlockSpec(memory_space=pl.ANY)],
        out_specs=out_spec, out_shape=out_shape,
        input_output_aliases={2: 0},
    )(packed, W, prev)


def kernel(input, table, W):
    n_batch, n_field = input.shape
    B = n_batch * n_field
    idx = jnp.pad(input.astype(jnp.int32), ((0, 0), (0, CH - n_field)))
    rows8 = -(-table.shape[0] // 8) * 8
    table_pad = jnp.pad(table, ((0, rows8 - table.shape[0]),
                                (0, EMB - BOTTLENECK)))
    table_view = table_pad.reshape(4 * rows8, BOTTLENECK)
    Bh = B // 2
    out_fm = None
    for h in range(2):
        flat_rows = _make_gather(Bh, n_field, h * (n_batch // 2))(
            idx, table_view)
        packed = flat_rows.reshape(Bh // 4, 4 * BOTTLENECK)
        out_fm = _unpack_project(packed, W, n_batch, n_field, h, out_fm)
    return jnp.transpose(out_fm, (1, 0, 2))
